# baseline (speedup 1.0000x reference)
import functools

import jax
import jax.numpy as jnp
from jax import lax
from jax.experimental import pallas as pl
from jax.experimental.pallas import tpu as pltpu
from jax.experimental.pallas import tpu_sc as plsc

_LANES = 16
_CHUNK = 128
_CBLK = 32768


def _tc_score_body(t_ref, w_ref, s_ref):
    s_ref[...] = jnp.dot(w_ref[...], t_ref[...],
                         preferred_element_type=jnp.float32)[0]


@functools.lru_cache(maxsize=None)
def _make_tc_score(n_rows, d):
    grid = (n_rows + _CBLK - 1) // _CBLK
    return pl.pallas_call(
        _tc_score_body,
        grid=(grid,),
        in_specs=[
            pl.BlockSpec((d, _CBLK), lambda i: (0, i)),
            pl.BlockSpec((8, d), lambda i: (0, 0)),
        ],
        out_specs=pl.BlockSpec((_CBLK,), lambda i: (i,)),
        out_shape=jax.ShapeDtypeStruct((n_rows,), jnp.float32),
    )


def _sc_params():
    return pltpu.CompilerParams(
        needs_layout_passes=False, use_tc_tiling_on_sc=True)


@functools.lru_cache(maxsize=None)
def _make_sc_gather1(B, n_cores, n_subcores):
    NW = n_cores * n_subcores
    per_w = B // NW
    n_chunks = per_w // _CHUNK
    mesh = plsc.VectorSubcoreMesh(core_axis_name="c", subcore_axis_name="s")

    @functools.partial(
        pl.kernel,
        out_type=jax.ShapeDtypeStruct((B,), jnp.float32),
        mesh=mesh,
        scratch_types=[
            pltpu.VMEM((per_w,), jnp.int32),
            pltpu.VMEM((per_w,), jnp.float32),
            pltpu.SemaphoreType.DMA,
        ],
        compiler_params=_sc_params(),
    )
    def sc1(idx_hbm, s_hbm, out_hbm, idx_v, vals, sem):
        wid = lax.axis_index("s") * n_cores + lax.axis_index("c")
        base = wid * per_w
        pltpu.sync_copy(idx_hbm.at[pl.ds(base, per_w)], idx_v)
        copies = []
        for k in range(n_chunks):
            sl = pl.ds(k * _CHUNK, _CHUNK)
            copies.append(
                pltpu.async_copy(s_hbm.at[idx_v.at[sl]], vals.at[sl], sem))
        for c in copies:
            c.wait()
        pltpu.sync_copy(vals, out_hbm.at[pl.ds(base, per_w)])

    return sc1


@functools.lru_cache(maxsize=None)
def _make_sc_gather2(B, n_cores, n_subcores):
    NW = n_cores * n_subcores
    per_w = B // NW
    n_chunks = per_w // _CHUNK
    mesh = plsc.VectorSubcoreMesh(core_axis_name="c", subcore_axis_name="s")

    @functools.partial(
        pl.kernel,
        out_type=jax.ShapeDtypeStruct((B,), jnp.float32),
        mesh=mesh,
        scratch_types=[
            pltpu.VMEM((per_w,), jnp.int32),
            pltpu.VMEM((per_w,), jnp.float32),
            pltpu.VMEM((per_w,), jnp.float32),
            pltpu.VMEM((_LANES,), jnp.float32),
            pltpu.SemaphoreType.DMA,
        ],
        compiler_params=_sc_params(),
    )
    def sc2(idx_hbm, s_hbm, part_hbm, bb_hbm, out_hbm,
            idx_v, vals, part_v, bv, sem):
        wid = lax.axis_index("s") * n_cores + lax.axis_index("c")
        base = wid * per_w
        pltpu.sync_copy(idx_hbm.at[pl.ds(base, per_w)], idx_v)
        pltpu.sync_copy(part_hbm.at[pl.ds(base, per_w)], part_v)
        pltpu.sync_copy(bb_hbm, bv)
        copies = []
        for k in range(n_chunks):
            sl = pl.ds(k * _CHUNK, _CHUNK)
            copies.append(
                pltpu.async_copy(s_hbm.at[idx_v.at[sl]], vals.at[sl], sem))
        for c in copies:
            c.wait()
        b_s = bv[pl.ds(0, _LANES)][0]
        for g in range(per_w // _LANES):
            sl = pl.ds(g * _LANES, _LANES)
            vals[sl] = vals[sl] + part_v[sl] + b_s
        pltpu.sync_copy(vals, out_hbm.at[pl.ds(base, per_w)])

    return sc2


def kernel(users, posts, user_table, post_table, W, b):
    B = users.shape[0]
    n_rows, d = user_table.shape
    info = plsc.get_sparse_core_info()
    nc, ns = info.num_cores, info.num_subcores

    wv = W.reshape(2 * d)
    wu8 = jnp.broadcast_to(wv[0:d].reshape(1, d), (8, d))
    wp8 = jnp.broadcast_to(wv[d:2 * d].reshape(1, d), (8, d))
    bb = jnp.broadcast_to(b, (_LANES,))

    tc_score = _make_tc_score(n_rows, d)
    su = tc_score(user_table.T, wu8)
    gu = _make_sc_gather1(B, nc, ns)(users.astype(jnp.int32), su)
    sp = tc_score(post_table.T, wp8)
    out = _make_sc_gather2(B, nc, ns)(
        posts.astype(jnp.int32), sp, gu, bb)
    return out.reshape(B, 1)


# speedup vs baseline: 1.1407x; 1.1407x over previous
import functools

import jax
import jax.numpy as jnp
from jax import lax
from jax.experimental import pallas as pl
from jax.experimental.pallas import tpu as pltpu
from jax.experimental.pallas import tpu_sc as plsc

_LANES = 16
_CHUNK = 128
_CBLK = 32768


def _tc_scores_body(tu_ref, tp_ref, w_ref, su_ref, sp_ref):
    d = tu_ref.shape[0]
    wu = w_ref[:, 0:d]
    wp = w_ref[:, d:2 * d]
    b = w_ref[0, 2 * d]
    su_ref[...] = jnp.dot(wu, tu_ref[...],
                          preferred_element_type=jnp.float32)[0] + b
    sp_ref[...] = jnp.dot(wp, tp_ref[...],
                          preferred_element_type=jnp.float32)[0]


@functools.lru_cache(maxsize=None)
def _make_tc_scores(n_rows, d):
    grid = (n_rows + _CBLK - 1) // _CBLK
    return pl.pallas_call(
        _tc_scores_body,
        grid=(grid,),
        in_specs=[
            pl.BlockSpec((d, _CBLK), lambda i: (0, i)),
            pl.BlockSpec((d, _CBLK), lambda i: (0, i)),
            pl.BlockSpec((8, 2 * d + 128), lambda i: (0, 0)),
        ],
        out_specs=[
            pl.BlockSpec((_CBLK,), lambda i: (i,)),
            pl.BlockSpec((_CBLK,), lambda i: (i,)),
        ],
        out_shape=[
            jax.ShapeDtypeStruct((n_rows,), jnp.float32),
            jax.ShapeDtypeStruct((n_rows,), jnp.float32),
        ],
    )


@functools.lru_cache(maxsize=None)
def _make_sc_gather(B, n_cores, n_subcores):
    NW = n_cores * n_subcores
    per_w = B // NW
    n_chunks = per_w // _CHUNK
    mesh = plsc.VectorSubcoreMesh(core_axis_name="c", subcore_axis_name="s")

    @functools.partial(
        pl.kernel,
        out_type=jax.ShapeDtypeStruct((B,), jnp.float32),
        mesh=mesh,
        scratch_types=[
            pltpu.VMEM((per_w,), jnp.int32),
            pltpu.VMEM((per_w,), jnp.int32),
            pltpu.VMEM((per_w,), jnp.float32),
            pltpu.VMEM((per_w,), jnp.float32),
            pltpu.SemaphoreType.DMA,
        ],
        compiler_params=pltpu.CompilerParams(
            needs_layout_passes=False, use_tc_tiling_on_sc=True),
    )
    def sc_kernel(users_hbm, posts_hbm, su_hbm, sp_hbm, out_hbm,
                  idx_u, idx_p, vu, vp, sem):
        wid = lax.axis_index("s") * n_cores + lax.axis_index("c")
        base = wid * per_w
        pltpu.sync_copy(users_hbm.at[pl.ds(base, per_w)], idx_u)
        pltpu.sync_copy(posts_hbm.at[pl.ds(base, per_w)], idx_p)

        copies = []
        for k in range(n_chunks):
            sl = pl.ds(k * _CHUNK, _CHUNK)
            copies.append(
                pltpu.async_copy(su_hbm.at[idx_u.at[sl]], vu.at[sl], sem))
            copies.append(
                pltpu.async_copy(sp_hbm.at[idx_p.at[sl]], vp.at[sl], sem))
        for c in copies:
            c.wait()

        for g in range(per_w // _LANES):
            sl = pl.ds(g * _LANES, _LANES)
            vu[sl] = vu[sl] + vp[sl]
        pltpu.sync_copy(vu, out_hbm.at[pl.ds(base, per_w)])

    return sc_kernel


def kernel(users, posts, user_table, post_table, W, b):
    B = users.shape[0]
    n_rows, d = user_table.shape
    info = plsc.get_sparse_core_info()

    wrow = jnp.concatenate([W.reshape(2 * d), b.reshape(1),
                            jnp.zeros((127,), jnp.float32)])
    w8 = jnp.broadcast_to(wrow.reshape(1, 2 * d + 128), (8, 2 * d + 128))
    su, sp = _make_tc_scores(n_rows, d)(user_table.T, post_table.T, w8)

    out = _make_sc_gather(B, info.num_cores, info.num_subcores)(
        users.astype(jnp.int32), posts.astype(jnp.int32), su, sp)
    return out.reshape(B, 1)


# speedup vs baseline: 1.1426x; 1.0017x over previous
import functools

import jax
import jax.numpy as jnp
from jax import lax
from jax.experimental import pallas as pl
from jax.experimental.pallas import tpu as pltpu
from jax.experimental.pallas import tpu_sc as plsc

_LANES = 16
_CHUNK = 128
_CBLK = 28672


def _tc_scores_body(tu_ref, tp_ref, w_ref, su_ref, sp_ref):
    d = tu_ref.shape[0]
    wu = w_ref[:, 0:d]
    wp = w_ref[:, d:2 * d]
    b = w_ref[0, 2 * d]
    su_ref[...] = jnp.dot(wu, tu_ref[...],
                          preferred_element_type=jnp.float32)[0] + b
    sp_ref[...] = jnp.dot(wp, tp_ref[...],
                          preferred_element_type=jnp.float32)[0]


@functools.lru_cache(maxsize=None)
def _make_tc_scores(n_rows, d):
    grid = (n_rows + _CBLK - 1) // _CBLK
    return pl.pallas_call(
        _tc_scores_body,
        grid=(grid,),
        in_specs=[
            pl.BlockSpec((d, _CBLK), lambda i: (0, i)),
            pl.BlockSpec((d, _CBLK), lambda i: (0, i)),
            pl.BlockSpec((8, 2 * d + 128), lambda i: (0, 0)),
        ],
        out_specs=[
            pl.BlockSpec((_CBLK,), lambda i: (i,)),
            pl.BlockSpec((_CBLK,), lambda i: (i,)),
        ],
        out_shape=[
            jax.ShapeDtypeStruct((n_rows,), jnp.float32),
            jax.ShapeDtypeStruct((n_rows,), jnp.float32),
        ],
    )


@functools.lru_cache(maxsize=None)
def _make_sc_gather(B, n_cores, n_subcores):
    NW = n_cores * n_subcores
    per_w = B // NW
    n_chunks = per_w // _CHUNK
    mesh = plsc.VectorSubcoreMesh(core_axis_name="c", subcore_axis_name="s")

    @functools.partial(
        pl.kernel,
        out_type=jax.ShapeDtypeStruct((B,), jnp.float32),
        mesh=mesh,
        scratch_types=[
            pltpu.VMEM((per_w,), jnp.int32),
            pltpu.VMEM((per_w,), jnp.int32),
            pltpu.VMEM((per_w,), jnp.float32),
            pltpu.VMEM((per_w,), jnp.float32),
            pltpu.SemaphoreType.DMA,
        ],
        compiler_params=pltpu.CompilerParams(
            needs_layout_passes=False, use_tc_tiling_on_sc=True),
    )
    def sc_kernel(users_hbm, posts_hbm, su_hbm, sp_hbm, out_hbm,
                  idx_u, idx_p, vu, vp, sem):
        wid = lax.axis_index("s") * n_cores + lax.axis_index("c")
        base = wid * per_w
        pltpu.sync_copy(users_hbm.at[pl.ds(base, per_w)], idx_u)
        pltpu.sync_copy(posts_hbm.at[pl.ds(base, per_w)], idx_p)

        copies = []
        for k in range(n_chunks):
            sl = pl.ds(k * _CHUNK, _CHUNK)
            copies.append(
                pltpu.async_copy(su_hbm.at[idx_u.at[sl]], vu.at[sl], sem))
            copies.append(
                pltpu.async_copy(sp_hbm.at[idx_p.at[sl]], vp.at[sl], sem))
        for c in copies:
            c.wait()

        for g in range(per_w // _LANES):
            sl = pl.ds(g * _LANES, _LANES)
            vu[sl] = vu[sl] + vp[sl]
        pltpu.sync_copy(vu, out_hbm.at[pl.ds(base, per_w)])

    return sc_kernel


def kernel(users, posts, user_table, post_table, W, b):
    B = users.shape[0]
    n_rows, d = user_table.shape
    info = plsc.get_sparse_core_info()

    wrow = jnp.concatenate([W.reshape(2 * d), b.reshape(1),
                            jnp.zeros((127,), jnp.float32)])
    w8 = jnp.broadcast_to(wrow.reshape(1, 2 * d + 128), (8, 2 * d + 128))
    su, sp = _make_tc_scores(n_rows, d)(user_table.T, post_table.T, w8)

    out = _make_sc_gather(B, info.num_cores, info.num_subcores)(
        users.astype(jnp.int32), posts.astype(jnp.int32), su, sp)
    return out.reshape(B, 1)
